# baseline (speedup 1.0000x reference)
import functools

import jax
import jax.numpy as jnp
from jax import lax
from jax.experimental import pallas as pl
from jax.experimental.pallas import tpu as pltpu
from jax.experimental.pallas import tpu_sc as plsc

F32 = jnp.float32
I32 = jnp.int32

D = 128
CH = 64
NSUB = 16
NCORE = 2
NWORK = NSUB * NCORE
TAGROWS = 80

NT = 5000
ND = 10000
NT_PAD = 5120
ND_PAD = 10240
RT_T = 5376
RT_D = 10496

E_WT = 80000
E_WD = 320000
E_TD = 80000


def _sc_body(feat_word, feat_topic, pk_wt, pk_wd, pk_td,
             out_wt, out_wd, out_td,
             acc, rows0, rows1, pk0, pk1, pk2, pk3, zeros, hzeros,
             dstidx0, dstidx1, tsw, tdeg, swidx, degidx,
             gsem0, gsem1, ssem0, ssem1, psem0, psem1, psem2, psem3):
    c = lax.axis_index("c")
    s = lax.axis_index("s")
    wid = c * NSUB + s
    lane = lax.iota(I32, 16)
    zero16 = jnp.zeros((16,), F32)
    ones16 = jnp.ones((16,), F32)
    rows_bufs = (rows0, rows1)
    pk_bufs = (pk0, pk1, pk2, pk3)
    dstix = (dstidx0, dstidx1)
    gsems = (gsem0, gsem1)
    ssems = (ssem0, ssem1)
    psems = (psem0, psem1, psem2, psem3)

    def zfill(r, carry):
        for j in range(D // 16):
            zeros[r, pl.ds(j * 16, 16)] = zero16
        return carry
    lax.fori_loop(0, 48, zfill, 0)

    pltpu.sync_copy(zeros.at[pl.ds(0, 41)], hzeros.at[pl.ds(s * 41, 41)])
    plsc.subcore_barrier()

    def do_phase(feat, pk, n_chunks, r_feat, r_total, out):
        rows_per_sub = r_total // NSUB

        def ifill(k, carry):
            swidx[pl.ds(k * 16, 16)] = r_feat + k * 16 + lane
            degidx[pl.ds(k * 16, 16)] = r_feat + TAGROWS + k * 16 + lane
            return carry
        lax.fori_loop(0, TAGROWS // 16, ifill, 0)

        def tz(g, carry):
            for j in range(D // 16):
                tsw[g, pl.ds(j * 16, 16)] = zero16
                tdeg[g, pl.ds(j * 16, 16)] = zero16
            return carry
        lax.fori_loop(0, TAGROWS, tz, 0)

        pltpu.sync_copy(hzeros.at[pl.ds(0, rows_per_sub)],
                        acc.at[pl.ds(s * rows_per_sub, rows_per_sub)])
        plsc.subcore_barrier()

        start_c = (wid * n_chunks) // NWORK
        cnt = ((wid + 1) * n_chunks) // NWORK - start_c

        def pk_start(j, b):
            pltpu.async_copy(pk.at[start_c + j], pk_bufs[b], psems[b])

        def pk_wait(b):
            pltpu.make_async_copy(pk.at[0], pk_bufs[b], psems[b]).wait()

        def g_start(pb, rb):
            pltpu.async_copy(feat.at[pk_bufs[pb].at[0]], rows_bufs[rb],
                             gsems[rb])

        def g_wait(rb):
            pltpu.make_async_copy(feat.at[pk_bufs[0].at[0]], rows_bufs[rb],
                                  gsems[rb]).wait()

        def s_start(b):
            pltpu.async_copy(rows_bufs[b], acc.at[dstix[b]], ssems[b],
                             add=True)

        def s_wait(b):
            pltpu.make_async_copy(rows_bufs[b], acc.at[dstix[b]],
                                  ssems[b]).wait()

        for q in range(4):
            @pl.when(q < cnt)
            def _():
                pk_start(q, q)

        @pl.when(cnt > 0)
        def _():
            pk_wait(0)
            g_start(0, 0)

        def quad_body(p, carry):
            for k in range(4):
                rb = k % 2
                pb = k
                j = 4 * p + k

                @pl.when(j < cnt)
                def _():
                    g_wait(rb)

                    @pl.when(j + 1 < cnt)
                    def _():
                        pk_wait((k + 1) % 4)

                        @pl.when(j >= 1)
                        def _():
                            s_wait(1 - rb)
                        g_start((k + 1) % 4, 1 - rb)

                    def gbody(g, carry2):
                        d16 = pk_bufs[pb][1, pl.ds(g * 16, 16)]
                        w16 = plsc.bitcast(pk_bufs[pb][2, pl.ds(g * 16, 16)],
                                           F32)
                        dstix[rb][pl.ds(g * 16, 16)] = d16
                        dr = lax.shift_right_logical(d16, 7)
                        dl = lax.bitwise_and(d16, 127)
                        plsc.addupdate_scatter(tsw, [dr, dl], w16)
                        plsc.addupdate_scatter(tdeg, [dr, dl], ones16)
                        for k2 in range(16):
                            r = g * 16 + k2
                            wspl = jnp.full((16,), w16[k2], dtype=F32)
                            for j2 in range(D // 16):
                                rows_bufs[rb][r, pl.ds(j2 * 16, 16)] = (
                                    rows_bufs[rb][r, pl.ds(j2 * 16, 16)]
                                    * wspl)
                        return carry2
                    lax.fori_loop(0, CH // 16, gbody, 0)
                    s_start(rb)

                    @pl.when(j + 4 < cnt)
                    def _():
                        pk_start(j + 4, pb)
            return carry
        lax.fori_loop(0, (cnt + 3) // 4, quad_body, 0)

        @pl.when(cnt >= 1)
        def _():
            s_wait(0)

        @pl.when(cnt >= 2)
        def _():
            s_wait(1)

        pltpu.sync_copy(tsw, acc.at[swidx], add=True)
        pltpu.sync_copy(tdeg, acc.at[degidx], add=True)
        plsc.subcore_barrier()

        pltpu.sync_copy(acc.at[pl.ds(s * rows_per_sub, rows_per_sub)],
                        out.at[pl.ds(c * r_total + s * rows_per_sub,
                                     rows_per_sub)])
        plsc.subcore_barrier()

    do_phase(feat_word, pk_wt, E_WT // CH, NT_PAD, RT_T, out_wt)
    do_phase(feat_word, pk_wd, E_WD // CH, ND_PAD, RT_D, out_wd)
    do_phase(feat_topic, pk_td, E_TD // CH, ND_PAD, RT_D, out_td)


_sc_agg = functools.partial(
    pl.kernel,
    out_type=[
        jax.ShapeDtypeStruct((NCORE * RT_T, D), F32),
        jax.ShapeDtypeStruct((NCORE * RT_D, D), F32),
        jax.ShapeDtypeStruct((NCORE * RT_D, D), F32),
    ],
    mesh=plsc.VectorSubcoreMesh(core_axis_name="c", subcore_axis_name="s"),
    compiler_params=pltpu.CompilerParams(needs_layout_passes=False),
    scratch_types=[
        pltpu.VMEM_SHARED((RT_D, D), F32),
        pltpu.VMEM((CH, D), F32),
        pltpu.VMEM((CH, D), F32),
        pltpu.VMEM((3, CH), I32),
        pltpu.VMEM((3, CH), I32),
        pltpu.VMEM((3, CH), I32),
        pltpu.VMEM((3, CH), I32),
        pltpu.VMEM((48, D), F32),
        pltpu.HBM((656, D), F32),
        pltpu.VMEM((CH,), I32),
        pltpu.VMEM((CH,), I32),
        pltpu.VMEM((TAGROWS, D), F32),
        pltpu.VMEM((TAGROWS, D), F32),
        pltpu.VMEM((TAGROWS,), I32),
        pltpu.VMEM((TAGROWS,), I32),
        pltpu.SemaphoreType.DMA,
        pltpu.SemaphoreType.DMA,
        pltpu.SemaphoreType.DMA,
        pltpu.SemaphoreType.DMA,
        pltpu.SemaphoreType.DMA,
        pltpu.SemaphoreType.DMA,
        pltpu.SemaphoreType.DMA,
        pltpu.SemaphoreType.DMA,
    ],
)(_sc_body)


def _fin_topic(pf_ref, psw_ref, pdeg_ref, wT_ref, b_ref, o_ref):
    a = pf_ref[0] + pf_ref[1]
    sw = psw_ref[0] + psw_ref[1]
    deg = pdeg_ref[0] + pdeg_ref[1]
    h = jnp.dot(a, wT_ref[...], preferred_element_type=F32) + sw * b_ref[...]
    o_ref[...] = h / jnp.maximum(deg, 1.0)


def _fin_doc(pfwd_ref, pswwd_ref, pdegwd_ref,
             pftd_ref, pswtd_ref, pdegtd_ref,
             wdT_ref, bwd_ref, tdT_ref, btd_ref, o_ref):
    awd = pfwd_ref[0] + pfwd_ref[1]
    swwd = pswwd_ref[0] + pswwd_ref[1]
    degwd = pdegwd_ref[0] + pdegwd_ref[1]
    atd = pftd_ref[0] + pftd_ref[1]
    swtd = pswtd_ref[0] + pswtd_ref[1]
    degtd = pdegtd_ref[0] + pdegtd_ref[1]
    hwd = (jnp.dot(awd, wdT_ref[...], preferred_element_type=F32)
           + swwd * bwd_ref[...]) / jnp.maximum(degwd, 1.0)
    htd = (jnp.dot(atd, tdT_ref[...], preferred_element_type=F32)
           + swtd * btd_ref[...]) / jnp.maximum(degtd, 1.0)
    o_ref[...] = jnp.maximum(hwd + htd, 0.0)


def _pack(src, dst, w):
    n = src.shape[0] // CH
    return jnp.stack([src.astype(I32).reshape(n, CH),
                      dst.astype(I32).reshape(n, CH),
                      lax.bitcast_convert_type(w.astype(F32),
                                               I32).reshape(n, CH)], axis=1)


def _unpack(p, r_total, n_pad):
    p = p.reshape(NCORE, r_total, D)
    tr = n_pad // D
    sw = p[:, n_pad:n_pad + tr].reshape(NCORE, n_pad, 1)
    deg = p[:, n_pad + TAGROWS:n_pad + TAGROWS + tr].reshape(NCORE, n_pad, 1)
    return p, sw, deg


def kernel(feat_word, feat_topic, feat_doc,
           wt_src, wt_dst, wd_src, wd_dst, td_src, td_dst,
           wt_weight, wd_weight, td_weight,
           W_wt, b_wt, W_wd, b_wd, W_td, b_td):
    p_wt, p_wd, p_td = _sc_agg(
        feat_word, feat_topic,
        _pack(wt_src, wt_dst, wt_weight),
        _pack(wd_src, wd_dst, wd_weight),
        _pack(td_src, td_dst, td_weight))

    f_wt, sw_wt, dg_wt = _unpack(p_wt, RT_T, NT_PAD)
    f_wd, sw_wd, dg_wd = _unpack(p_wd, RT_D, ND_PAD)
    f_td, sw_td, dg_td = _unpack(p_td, RT_D, ND_PAD)

    BT = 512
    full = lambda i: (0, 0)
    h_topic = pl.pallas_call(
        _fin_topic,
        grid=(NT_PAD // BT,),
        in_specs=[
            pl.BlockSpec((NCORE, BT, D), lambda i: (0, i, 0)),
            pl.BlockSpec((NCORE, BT, 1), lambda i: (0, i, 0)),
            pl.BlockSpec((NCORE, BT, 1), lambda i: (0, i, 0)),
            pl.BlockSpec((D, D), full),
            pl.BlockSpec((1, D), full),
        ],
        out_specs=pl.BlockSpec((BT, D), lambda i: (i, 0)),
        out_shape=jax.ShapeDtypeStruct((NT, D), F32),
    )(f_wt, sw_wt, dg_wt, W_wt.T, b_wt.reshape(1, D))

    h_doc = pl.pallas_call(
        _fin_doc,
        grid=(ND_PAD // BT,),
        in_specs=[
            pl.BlockSpec((NCORE, BT, D), lambda i: (0, i, 0)),
            pl.BlockSpec((NCORE, BT, 1), lambda i: (0, i, 0)),
            pl.BlockSpec((NCORE, BT, 1), lambda i: (0, i, 0)),
            pl.BlockSpec((NCORE, BT, D), lambda i: (0, i, 0)),
            pl.BlockSpec((NCORE, BT, 1), lambda i: (0, i, 0)),
            pl.BlockSpec((NCORE, BT, 1), lambda i: (0, i, 0)),
            pl.BlockSpec((D, D), full),
            pl.BlockSpec((1, D), full),
            pl.BlockSpec((D, D), full),
            pl.BlockSpec((1, D), full),
        ],
        out_specs=pl.BlockSpec((BT, D), lambda i: (i, 0)),
        out_shape=jax.ShapeDtypeStruct((ND, D), F32),
    )(f_wd, sw_wd, dg_wd, f_td, sw_td, dg_td,
      W_wd.T, b_wd.reshape(1, D), W_td.T, b_td.reshape(1, D))

    return (feat_word, h_topic, h_doc)


# speedup vs baseline: 1.0005x; 1.0005x over previous
import functools

import jax
import jax.numpy as jnp
from jax import lax
from jax.experimental import pallas as pl
from jax.experimental.pallas import tpu as pltpu
from jax.experimental.pallas import tpu_sc as plsc

F32 = jnp.float32
I32 = jnp.int32

D = 128
CH = 64
NSUB = 16
NCORE = 2
NWORK = NSUB * NCORE
TAGROWS = 80

NT = 5000
ND = 10000
NT_PAD = 5120
ND_PAD = 10240
RT_T = 5376
RT_D = 10496

E_WT = 80000
E_WD = 320000
E_TD = 80000


def _sc_body(feat_word, feat_topic, pk_wt, pk_wd, pk_td,
             out_wt, out_wd, out_td,
             acc, rows0, rows1, pk0, pk1, zeros, hzeros,
             dstidx0, dstidx1, tsw, tdeg, swidx, degidx,
             gsem0, gsem1, ssem0, ssem1, psem0, psem1):
    c = lax.axis_index("c")
    s = lax.axis_index("s")
    wid = c * NSUB + s
    lane = lax.iota(I32, 16)
    zero16 = jnp.zeros((16,), F32)
    ones16 = jnp.ones((16,), F32)
    rows_bufs = (rows0, rows1)
    pk_bufs = (pk0, pk1)
    dstix = (dstidx0, dstidx1)
    gsems = (gsem0, gsem1)
    ssems = (ssem0, ssem1)
    psems = (psem0, psem1)

    def zfill(r, carry):
        for j in range(D // 16):
            zeros[r, pl.ds(j * 16, 16)] = zero16
        return carry
    lax.fori_loop(0, 64, zfill, 0)

    pltpu.sync_copy(zeros.at[pl.ds(0, 41)], hzeros.at[pl.ds(s * 41, 41)])
    plsc.subcore_barrier()

    def do_phase(feat, pk, n_chunks, r_feat, r_total, out):
        rows_per_sub = r_total // NSUB

        def ifill(k, carry):
            swidx[pl.ds(k * 16, 16)] = r_feat + k * 16 + lane
            degidx[pl.ds(k * 16, 16)] = r_feat + TAGROWS + k * 16 + lane
            return carry
        lax.fori_loop(0, TAGROWS // 16, ifill, 0)

        def tz(g, carry):
            for j in range(D // 16):
                tsw[g, pl.ds(j * 16, 16)] = zero16
                tdeg[g, pl.ds(j * 16, 16)] = zero16
            return carry
        lax.fori_loop(0, TAGROWS, tz, 0)

        pltpu.sync_copy(hzeros.at[pl.ds(0, rows_per_sub)],
                        acc.at[pl.ds(s * rows_per_sub, rows_per_sub)])
        plsc.subcore_barrier()

        start_c = (wid * n_chunks) // NWORK
        cnt = ((wid + 1) * n_chunks) // NWORK - start_c

        def pk_start(j, b):
            pltpu.async_copy(pk.at[start_c + j], pk_bufs[b], psems[b])

        def pk_wait(b):
            pltpu.make_async_copy(pk.at[0], pk_bufs[b], psems[b]).wait()

        def g_start(b):
            pltpu.async_copy(feat.at[pk_bufs[b].at[0]], rows_bufs[b],
                             gsems[b])

        def g_wait(b):
            pltpu.make_async_copy(feat.at[pk_bufs[b].at[0]], rows_bufs[b],
                                  gsems[b]).wait()

        def s_start(b):
            pltpu.async_copy(rows_bufs[b], acc.at[dstix[b]], ssems[b],
                             add=True)

        def s_wait(b):
            pltpu.make_async_copy(rows_bufs[b], acc.at[dstix[b]],
                                  ssems[b]).wait()

        @pl.when(cnt > 0)
        def _():
            pk_start(0, 0)

            @pl.when(cnt > 1)
            def _():
                pk_start(1, 1)
            pk_wait(0)
            g_start(0)

        def pair_body(p, carry):
            for k in (0, 1):
                b = k
                j = 2 * p + k

                @pl.when(j < cnt)
                def _():
                    g_wait(b)

                    @pl.when(j + 1 < cnt)
                    def _():
                        pk_wait(1 - b)

                        @pl.when(j >= 1)
                        def _():
                            s_wait(1 - b)
                        g_start(1 - b)

                    def gbody(g, carry2):
                        d16 = pk_bufs[b][1, pl.ds(g * 16, 16)]
                        w16 = plsc.bitcast(pk_bufs[b][2, pl.ds(g * 16, 16)],
                                           F32)
                        dstix[b][pl.ds(g * 16, 16)] = d16
                        dr = lax.shift_right_logical(d16, 7)
                        dl = lax.bitwise_and(d16, 127)
                        plsc.addupdate_scatter(tsw, [dr, dl], w16)
                        plsc.addupdate_scatter(tdeg, [dr, dl], ones16)
                        for k2 in range(16):
                            r = g * 16 + k2
                            wspl = jnp.full((16,), w16[k2], dtype=F32)
                            for j2 in range(D // 16):
                                rows_bufs[b][r, pl.ds(j2 * 16, 16)] = (
                                    rows_bufs[b][r, pl.ds(j2 * 16, 16)] * wspl)
                        return carry2
                    lax.fori_loop(0, CH // 16, gbody, 0)
                    s_start(b)

                    @pl.when(j + 2 < cnt)
                    def _():
                        pk_start(j + 2, b)
            return carry
        lax.fori_loop(0, (cnt + 1) // 2, pair_body, 0)

        @pl.when(cnt >= 1)
        def _():
            s_wait(0)

        @pl.when(cnt >= 2)
        def _():
            s_wait(1)

        pltpu.sync_copy(tsw, acc.at[swidx], add=True)
        pltpu.sync_copy(tdeg, acc.at[degidx], add=True)
        plsc.subcore_barrier()

        pltpu.sync_copy(acc.at[pl.ds(s * rows_per_sub, rows_per_sub)],
                        out.at[pl.ds(c * r_total + s * rows_per_sub,
                                     rows_per_sub)])
        plsc.subcore_barrier()

    do_phase(feat_word, pk_wt, E_WT // CH, NT_PAD, RT_T, out_wt)
    do_phase(feat_word, pk_wd, E_WD // CH, ND_PAD, RT_D, out_wd)
    do_phase(feat_topic, pk_td, E_TD // CH, ND_PAD, RT_D, out_td)


_sc_agg = functools.partial(
    pl.kernel,
    out_type=[
        jax.ShapeDtypeStruct((NCORE * RT_T, D), F32),
        jax.ShapeDtypeStruct((NCORE * RT_D, D), F32),
        jax.ShapeDtypeStruct((NCORE * RT_D, D), F32),
    ],
    mesh=plsc.VectorSubcoreMesh(core_axis_name="c", subcore_axis_name="s"),
    compiler_params=pltpu.CompilerParams(needs_layout_passes=False),
    scratch_types=[
        pltpu.VMEM_SHARED((RT_D, D), F32),
        pltpu.VMEM((CH, D), F32),
        pltpu.VMEM((CH, D), F32),
        pltpu.VMEM((3, CH), I32),
        pltpu.VMEM((3, CH), I32),
        pltpu.VMEM((64, D), F32),
        pltpu.HBM((656, D), F32),
        pltpu.VMEM((CH,), I32),
        pltpu.VMEM((CH,), I32),
        pltpu.VMEM((TAGROWS, D), F32),
        pltpu.VMEM((TAGROWS, D), F32),
        pltpu.VMEM((TAGROWS,), I32),
        pltpu.VMEM((TAGROWS,), I32),
        pltpu.SemaphoreType.DMA,
        pltpu.SemaphoreType.DMA,
        pltpu.SemaphoreType.DMA,
        pltpu.SemaphoreType.DMA,
        pltpu.SemaphoreType.DMA,
        pltpu.SemaphoreType.DMA,
    ],
)(_sc_body)


def _fin_topic(pf_ref, psw_ref, pdeg_ref, wT_ref, b_ref, o_ref):
    a = pf_ref[0] + pf_ref[1]
    sw = psw_ref[0] + psw_ref[1]
    deg = pdeg_ref[0] + pdeg_ref[1]
    h = jnp.dot(a, wT_ref[...], preferred_element_type=F32) + sw * b_ref[...]
    o_ref[...] = h / jnp.maximum(deg, 1.0)


def _fin_doc(pfwd_ref, pswwd_ref, pdegwd_ref,
             pftd_ref, pswtd_ref, pdegtd_ref,
             wdT_ref, bwd_ref, tdT_ref, btd_ref, o_ref):
    awd = pfwd_ref[0] + pfwd_ref[1]
    swwd = pswwd_ref[0] + pswwd_ref[1]
    degwd = pdegwd_ref[0] + pdegwd_ref[1]
    atd = pftd_ref[0] + pftd_ref[1]
    swtd = pswtd_ref[0] + pswtd_ref[1]
    degtd = pdegtd_ref[0] + pdegtd_ref[1]
    hwd = (jnp.dot(awd, wdT_ref[...], preferred_element_type=F32)
           + swwd * bwd_ref[...]) / jnp.maximum(degwd, 1.0)
    htd = (jnp.dot(atd, tdT_ref[...], preferred_element_type=F32)
           + swtd * btd_ref[...]) / jnp.maximum(degtd, 1.0)
    o_ref[...] = jnp.maximum(hwd + htd, 0.0)


def _pack(src, dst, w):
    n = src.shape[0] // CH
    return jnp.stack([src.astype(I32).reshape(n, CH),
                      dst.astype(I32).reshape(n, CH),
                      lax.bitcast_convert_type(w.astype(F32),
                                               I32).reshape(n, CH)], axis=1)


def _unpack(p, r_total, n_pad):
    p = p.reshape(NCORE, r_total, D)
    tr = n_pad // D
    sw = p[:, n_pad:n_pad + tr].reshape(NCORE, n_pad, 1)
    deg = p[:, n_pad + TAGROWS:n_pad + TAGROWS + tr].reshape(NCORE, n_pad, 1)
    return p, sw, deg


def kernel(feat_word, feat_topic, feat_doc,
           wt_src, wt_dst, wd_src, wd_dst, td_src, td_dst,
           wt_weight, wd_weight, td_weight,
           W_wt, b_wt, W_wd, b_wd, W_td, b_td):
    p_wt, p_wd, p_td = _sc_agg(
        feat_word, feat_topic,
        _pack(wt_src, wt_dst, wt_weight),
        _pack(wd_src, wd_dst, wd_weight),
        _pack(td_src, td_dst, td_weight))

    f_wt, sw_wt, dg_wt = _unpack(p_wt, RT_T, NT_PAD)
    f_wd, sw_wd, dg_wd = _unpack(p_wd, RT_D, ND_PAD)
    f_td, sw_td, dg_td = _unpack(p_td, RT_D, ND_PAD)

    BT = 512
    full = lambda i: (0, 0)
    h_topic = pl.pallas_call(
        _fin_topic,
        grid=(NT_PAD // BT,),
        in_specs=[
            pl.BlockSpec((NCORE, BT, D), lambda i: (0, i, 0)),
            pl.BlockSpec((NCORE, BT, 1), lambda i: (0, i, 0)),
            pl.BlockSpec((NCORE, BT, 1), lambda i: (0, i, 0)),
            pl.BlockSpec((D, D), full),
            pl.BlockSpec((1, D), full),
        ],
        out_specs=pl.BlockSpec((BT, D), lambda i: (i, 0)),
        out_shape=jax.ShapeDtypeStruct((NT, D), F32),
    )(f_wt, sw_wt, dg_wt, W_wt.T, b_wt.reshape(1, D))

    h_doc = pl.pallas_call(
        _fin_doc,
        grid=(ND_PAD // BT,),
        in_specs=[
            pl.BlockSpec((NCORE, BT, D), lambda i: (0, i, 0)),
            pl.BlockSpec((NCORE, BT, 1), lambda i: (0, i, 0)),
            pl.BlockSpec((NCORE, BT, 1), lambda i: (0, i, 0)),
            pl.BlockSpec((NCORE, BT, D), lambda i: (0, i, 0)),
            pl.BlockSpec((NCORE, BT, 1), lambda i: (0, i, 0)),
            pl.BlockSpec((NCORE, BT, 1), lambda i: (0, i, 0)),
            pl.BlockSpec((D, D), full),
            pl.BlockSpec((1, D), full),
            pl.BlockSpec((D, D), full),
            pl.BlockSpec((1, D), full),
        ],
        out_specs=pl.BlockSpec((BT, D), lambda i: (i, 0)),
        out_shape=jax.ShapeDtypeStruct((ND, D), F32),
    )(f_wd, sw_wd, dg_wd, f_td, sw_td, dg_td,
      W_wd.T, b_wd.reshape(1, D), W_td.T, b_td.reshape(1, D))

    return (feat_word, h_topic, h_doc)


# speedup vs baseline: 1.0328x; 1.0322x over previous
import functools

import jax
import jax.numpy as jnp
from jax import lax
from jax.experimental import pallas as pl
from jax.experimental.pallas import tpu as pltpu
from jax.experimental.pallas import tpu_sc as plsc

F32 = jnp.float32
I32 = jnp.int32

D = 128
CH = 64
NSUB = 16
NCORE = 2
NWORK = NSUB * NCORE
TAGROWS = 80

NT = 5000
ND = 10000
NT_PAD = 5120
ND_PAD = 10240
RT_T = 5376
RT_D = 10496

E_WT = 80000
E_WD = 320000
E_TD = 80000


def _sc_body(feat_word, feat_topic, pk_wt, pk_wd, pk_td,
             out_wt, out_wd, out_td,
             acc, rows0, rows1, pk0, pk1, zeros,
             dstidx0, dstidx1, tsw, tdeg, swidx, degidx,
             gsem0, gsem1, ssem0, ssem1, psem0, psem1):
    c = lax.axis_index("c")
    s = lax.axis_index("s")
    wid = c * NSUB + s
    lane = lax.iota(I32, 16)
    zero16 = jnp.zeros((16,), F32)
    ones16 = jnp.ones((16,), F32)
    rows_bufs = (rows0, rows1)
    pk_bufs = (pk0, pk1)
    dstix = (dstidx0, dstidx1)
    gsems = (gsem0, gsem1)
    ssems = (ssem0, ssem1)
    psems = (psem0, psem1)

    def zfill(r, carry):
        for j in range(D // 16):
            zeros[r, pl.ds(j * 16, 16)] = zero16
        return carry
    lax.fori_loop(0, 64, zfill, 0)

    def do_phase(feat, pk, n_chunks, r_feat, r_total, out):
        rows_per_sub = r_total // NSUB

        def ifill(k, carry):
            swidx[pl.ds(k * 16, 16)] = r_feat + k * 16 + lane
            degidx[pl.ds(k * 16, 16)] = r_feat + TAGROWS + k * 16 + lane
            return carry
        lax.fori_loop(0, TAGROWS // 16, ifill, 0)

        def tz(g, carry):
            for j in range(D // 16):
                tsw[g, pl.ds(j * 16, 16)] = zero16
                tdeg[g, pl.ds(j * 16, 16)] = zero16
            return carry
        lax.fori_loop(0, TAGROWS, tz, 0)

        nslab = rows_per_sub // 64

        def zloop(k, carry):
            pltpu.async_copy(zeros,
                             acc.at[pl.ds(s * rows_per_sub + k * 64, 64)],
                             gsem0)
            return carry
        lax.fori_loop(0, nslab, zloop, 0)
        pltpu.async_copy(zeros.at[pl.ds(0, 16)],
                         acc.at[pl.ds(s * rows_per_sub + nslab * 64, 16)],
                         gsem0)

        def zdrain(k, carry):
            pltpu.make_async_copy(
                zeros, acc.at[pl.ds(s * rows_per_sub, 64)], gsem0).wait()
            return carry
        lax.fori_loop(0, nslab, zdrain, 0)
        pltpu.make_async_copy(zeros.at[pl.ds(0, 16)],
                              acc.at[pl.ds(s * rows_per_sub, 16)],
                              gsem0).wait()
        plsc.subcore_barrier()

        start_c = (wid * n_chunks) // NWORK
        cnt = ((wid + 1) * n_chunks) // NWORK - start_c

        def pk_start(j, b):
            pltpu.async_copy(pk.at[start_c + j], pk_bufs[b], psems[b])

        def pk_wait(b):
            pltpu.make_async_copy(pk.at[0], pk_bufs[b], psems[b]).wait()

        def g_start(b):
            pltpu.async_copy(feat.at[pk_bufs[b].at[0]], rows_bufs[b],
                             gsems[b])

        def g_wait(b):
            pltpu.make_async_copy(feat.at[pk_bufs[b].at[0]], rows_bufs[b],
                                  gsems[b]).wait()

        def s_start(b):
            pltpu.async_copy(rows_bufs[b], acc.at[dstix[b]], ssems[b],
                             add=True)

        def s_wait(b):
            pltpu.make_async_copy(rows_bufs[b], acc.at[dstix[b]],
                                  ssems[b]).wait()

        @pl.when(cnt > 0)
        def _():
            pk_start(0, 0)

            @pl.when(cnt > 1)
            def _():
                pk_start(1, 1)
            pk_wait(0)
            g_start(0)

        def pair_body(p, carry):
            for k in (0, 1):
                b = k
                j = 2 * p + k

                @pl.when(j < cnt)
                def _():
                    g_wait(b)

                    @pl.when(j + 1 < cnt)
                    def _():
                        pk_wait(1 - b)

                        @pl.when(j >= 1)
                        def _():
                            s_wait(1 - b)
                        g_start(1 - b)

                    def gbody(g, carry2):
                        d16 = pk_bufs[b][1, pl.ds(g * 16, 16)]
                        w16 = plsc.bitcast(pk_bufs[b][2, pl.ds(g * 16, 16)],
                                           F32)
                        dstix[b][pl.ds(g * 16, 16)] = d16
                        dr = lax.shift_right_logical(d16, 7)
                        dl = lax.bitwise_and(d16, 127)
                        plsc.addupdate_scatter(tsw, [dr, dl], w16)
                        plsc.addupdate_scatter(tdeg, [dr, dl], ones16)
                        for k2 in range(16):
                            r = g * 16 + k2
                            wspl = jnp.full((16,), w16[k2], dtype=F32)
                            for j2 in range(D // 16):
                                rows_bufs[b][r, pl.ds(j2 * 16, 16)] = (
                                    rows_bufs[b][r, pl.ds(j2 * 16, 16)] * wspl)
                        return carry2
                    lax.fori_loop(0, CH // 16, gbody, 0)
                    s_start(b)

                    @pl.when(j + 2 < cnt)
                    def _():
                        pk_start(j + 2, b)
            return carry
        lax.fori_loop(0, (cnt + 1) // 2, pair_body, 0)

        @pl.when(cnt >= 1)
        def _():
            s_wait(0)

        @pl.when(cnt >= 2)
        def _():
            s_wait(1)

        pltpu.sync_copy(tsw, acc.at[swidx], add=True)
        pltpu.sync_copy(tdeg, acc.at[degidx], add=True)
        plsc.subcore_barrier()

        pltpu.sync_copy(acc.at[pl.ds(s * rows_per_sub, rows_per_sub)],
                        out.at[pl.ds(c * r_total + s * rows_per_sub,
                                     rows_per_sub)])
        plsc.subcore_barrier()

    do_phase(feat_word, pk_wt, E_WT // CH, NT_PAD, RT_T, out_wt)
    do_phase(feat_word, pk_wd, E_WD // CH, ND_PAD, RT_D, out_wd)
    do_phase(feat_topic, pk_td, E_TD // CH, ND_PAD, RT_D, out_td)


_sc_agg = functools.partial(
    pl.kernel,
    out_type=[
        jax.ShapeDtypeStruct((NCORE * RT_T, D), F32),
        jax.ShapeDtypeStruct((NCORE * RT_D, D), F32),
        jax.ShapeDtypeStruct((NCORE * RT_D, D), F32),
    ],
    mesh=plsc.VectorSubcoreMesh(core_axis_name="c", subcore_axis_name="s"),
    compiler_params=pltpu.CompilerParams(needs_layout_passes=False),
    scratch_types=[
        pltpu.VMEM_SHARED((RT_D, D), F32),
        pltpu.VMEM((CH, D), F32),
        pltpu.VMEM((CH, D), F32),
        pltpu.VMEM((3, CH), I32),
        pltpu.VMEM((3, CH), I32),
        pltpu.VMEM((64, D), F32),
        pltpu.VMEM((CH,), I32),
        pltpu.VMEM((CH,), I32),
        pltpu.VMEM((TAGROWS, D), F32),
        pltpu.VMEM((TAGROWS, D), F32),
        pltpu.VMEM((TAGROWS,), I32),
        pltpu.VMEM((TAGROWS,), I32),
        pltpu.SemaphoreType.DMA,
        pltpu.SemaphoreType.DMA,
        pltpu.SemaphoreType.DMA,
        pltpu.SemaphoreType.DMA,
        pltpu.SemaphoreType.DMA,
        pltpu.SemaphoreType.DMA,
    ],
)(_sc_body)


def _fin_topic(pf_ref, psw_ref, pdeg_ref, wT_ref, b_ref, o_ref):
    a = pf_ref[0] + pf_ref[1]
    sw = psw_ref[0] + psw_ref[1]
    deg = pdeg_ref[0] + pdeg_ref[1]
    h = jnp.dot(a, wT_ref[...], preferred_element_type=F32) + sw * b_ref[...]
    o_ref[...] = h / jnp.maximum(deg, 1.0)


def _fin_doc(pfwd_ref, pswwd_ref, pdegwd_ref,
             pftd_ref, pswtd_ref, pdegtd_ref,
             wdT_ref, bwd_ref, tdT_ref, btd_ref, o_ref):
    awd = pfwd_ref[0] + pfwd_ref[1]
    swwd = pswwd_ref[0] + pswwd_ref[1]
    degwd = pdegwd_ref[0] + pdegwd_ref[1]
    atd = pftd_ref[0] + pftd_ref[1]
    swtd = pswtd_ref[0] + pswtd_ref[1]
    degtd = pdegtd_ref[0] + pdegtd_ref[1]
    hwd = (jnp.dot(awd, wdT_ref[...], preferred_element_type=F32)
           + swwd * bwd_ref[...]) / jnp.maximum(degwd, 1.0)
    htd = (jnp.dot(atd, tdT_ref[...], preferred_element_type=F32)
           + swtd * btd_ref[...]) / jnp.maximum(degtd, 1.0)
    o_ref[...] = jnp.maximum(hwd + htd, 0.0)


def _pack(src, dst, w):
    n = src.shape[0] // CH
    return jnp.stack([src.astype(I32).reshape(n, CH),
                      dst.astype(I32).reshape(n, CH),
                      lax.bitcast_convert_type(w.astype(F32),
                                               I32).reshape(n, CH)], axis=1)


def _unpack(p, r_total, n_pad):
    p = p.reshape(NCORE, r_total, D)
    tr = n_pad // D
    sw = p[:, n_pad:n_pad + tr].reshape(NCORE, n_pad, 1)
    deg = p[:, n_pad + TAGROWS:n_pad + TAGROWS + tr].reshape(NCORE, n_pad, 1)
    return p, sw, deg


def kernel(feat_word, feat_topic, feat_doc,
           wt_src, wt_dst, wd_src, wd_dst, td_src, td_dst,
           wt_weight, wd_weight, td_weight,
           W_wt, b_wt, W_wd, b_wd, W_td, b_td):
    p_wt, p_wd, p_td = _sc_agg(
        feat_word, feat_topic,
        _pack(wt_src, wt_dst, wt_weight),
        _pack(wd_src, wd_dst, wd_weight),
        _pack(td_src, td_dst, td_weight))

    f_wt, sw_wt, dg_wt = _unpack(p_wt, RT_T, NT_PAD)
    f_wd, sw_wd, dg_wd = _unpack(p_wd, RT_D, ND_PAD)
    f_td, sw_td, dg_td = _unpack(p_td, RT_D, ND_PAD)

    BT = 512
    full = lambda i: (0, 0)
    h_topic = pl.pallas_call(
        _fin_topic,
        grid=(NT_PAD // BT,),
        in_specs=[
            pl.BlockSpec((NCORE, BT, D), lambda i: (0, i, 0)),
            pl.BlockSpec((NCORE, BT, 1), lambda i: (0, i, 0)),
            pl.BlockSpec((NCORE, BT, 1), lambda i: (0, i, 0)),
            pl.BlockSpec((D, D), full),
            pl.BlockSpec((1, D), full),
        ],
        out_specs=pl.BlockSpec((BT, D), lambda i: (i, 0)),
        out_shape=jax.ShapeDtypeStruct((NT, D), F32),
    )(f_wt, sw_wt, dg_wt, W_wt.T, b_wt.reshape(1, D))

    h_doc = pl.pallas_call(
        _fin_doc,
        grid=(ND_PAD // BT,),
        in_specs=[
            pl.BlockSpec((NCORE, BT, D), lambda i: (0, i, 0)),
            pl.BlockSpec((NCORE, BT, 1), lambda i: (0, i, 0)),
            pl.BlockSpec((NCORE, BT, 1), lambda i: (0, i, 0)),
            pl.BlockSpec((NCORE, BT, D), lambda i: (0, i, 0)),
            pl.BlockSpec((NCORE, BT, 1), lambda i: (0, i, 0)),
            pl.BlockSpec((NCORE, BT, 1), lambda i: (0, i, 0)),
            pl.BlockSpec((D, D), full),
            pl.BlockSpec((1, D), full),
            pl.BlockSpec((D, D), full),
            pl.BlockSpec((1, D), full),
        ],
        out_specs=pl.BlockSpec((BT, D), lambda i: (i, 0)),
        out_shape=jax.ShapeDtypeStruct((ND, D), F32),
    )(f_wd, sw_wd, dg_wd, f_td, sw_td, dg_td,
      W_wd.T, b_wd.reshape(1, D), W_td.T, b_td.reshape(1, D))

    return (feat_word, h_topic, h_doc)
